# baseline (speedup 1.0000x reference)
import functools

import jax
import jax.numpy as jnp
from jax import lax
from jax.experimental import pallas as pl
from jax.experimental.pallas import tpu as pltpu
from jax.experimental.pallas import tpu_sc as plsc

_INFO = plsc.get_sparse_core_info()
_NC, _NS = _INFO.num_cores, _INFO.num_subcores
_NW = _NC * _NS
_NB = 2
_CT = 8


def _make_gather(B, T, V, D):
  b_per_w = B // _NW
  n_chunks = b_per_w // _NB
  seg = 24
  per_chunk = (_CT - 1) * _NB * seg + _NB * T
  n_per_w = n_chunks * per_chunk
  mesh = plsc.VectorSubcoreMesh(core_axis_name="c", subcore_axis_name="s")

  @functools.partial(
      pl.kernel,
      mesh=mesh,
      out_type=jax.ShapeDtypeStruct((B, T, D), jnp.float32),
      scratch_types=[
          pltpu.VMEM((2 * per_chunk,), jnp.int32),
          pltpu.VMEM((2, _NB, T, D), jnp.float32),
          pltpu.VMEM((2, _NB * T, 128), jnp.float32),
          pltpu.SemaphoreType.DMA,
          pltpu.SemaphoreType.DMA,
          pltpu.SemaphoreType.DMA,
          pltpu.SemaphoreType.DMA,
      ],
  )
  def gather_kernel(t128_hbm, sidx_hbm, out_hbm, sidx_v, rows_v, tail_v,
                    gsem0, gsem1, ssem0, ssem1):
    wid = lax.axis_index("s") * _NC + lax.axis_index("c")
    gsems = (gsem0, gsem1)
    ssems = (ssem0, ssem1)
    def fire_gathers(c, x, local, stage):
      if stage:
        pltpu.sync_copy(
            sidx_hbm.at[pl.ds(wid * n_per_w + (c // 2) * (2 * per_chunk),
                              2 * per_chunk)],
            sidx_v)
      o = local * per_chunk
      for bb in range(_NB):
        for ct in range(_CT - 1):
          pltpu.async_copy(
              t128_hbm.at[sidx_v.at[pl.ds(o + (ct * _NB + bb) * seg, T)]],
              rows_v.at[x, bb, pl.ds(0, T), pl.ds(ct * 128, 128)],
              gsems[x],
          )
      pltpu.async_copy(
          t128_hbm.at[sidx_v.at[pl.ds(o + (_CT - 1) * _NB * seg, _NB * T)]],
          tail_v.at[x], gsems[x])

    def wait_gathers(x):
      for bb in range(_NB):
        for ct in range(_CT - 1):
          pltpu.make_async_copy(
              t128_hbm.at[sidx_v.at[pl.ds((ct * _NB + bb) * seg, T)]],
              rows_v.at[x, bb, pl.ds(0, T), pl.ds(ct * 128, 128)],
              gsems[x],
          ).wait()
      pltpu.make_async_copy(
          t128_hbm.at[sidx_v.at[pl.ds(0, _NB * T)]], tail_v.at[x],
          gsems[x]).wait()

    def merge(x):
      def merge_row(t, _):
        for bb in range(_NB):
          for j in range(6):
            rows_v[x, bb, t, pl.ds(896 + 16 * j, 16)] = (
                tail_v[x, bb * T + t, pl.ds(16 * j, 16)])
          over = pl.multiple_of(t * 0 + 992, 16)
          rows_v[x, bb, t, pl.ds(over, 16)] = (
              tail_v[x, bb * T + t, pl.ds(96, 16)])
        return 0

      lax.fori_loop(0, T, merge_row, 0)

    def fire_scatter(c, x):
      b0 = wid * b_per_w + c * _NB
      pltpu.async_copy(rows_v.at[x], out_hbm.at[pl.ds(b0, _NB)], ssems[x])

    def wait_scatter(c, x):
      b0 = wid * b_per_w + c * _NB
      pltpu.make_async_copy(rows_v.at[x], out_hbm.at[pl.ds(b0, _NB)],
                            ssems[x]).wait()

    fire_gathers(0, 0, 0, True)
    wait_gathers(0)
    merge(0)
    fire_scatter(0, 0)
    fire_gathers(1, 1, 1, False)

    def body(p, _):
      j = 2 * p + 1
      wait_gathers(1)
      merge(1)
      fire_scatter(j, 1)
      wait_scatter(j - 1, 0)
      fire_gathers(j + 1, 0, 0, True)

      wait_gathers(0)
      merge(0)
      fire_scatter(j + 1, 0)
      wait_scatter(j, 1)
      fire_gathers(j + 2, 1, 1, False)
      return 0

    lax.fori_loop(0, (n_chunks - 2) // 2, body, 0)

    last = n_chunks - 1
    wait_gathers(1)
    merge(1)
    wait_scatter(last - 1, 0)
    fire_scatter(last, 1)
    wait_scatter(last, 1)

  return gather_kernel


def kernel(idx, table):
  B, T = idx.shape
  V, D = table.shape
  idx32 = idx.astype(jnp.int32)
  t128 = jnp.pad(table, ((0, 0), (0, _CT * 128 - D))).reshape(V * _CT, 128)
  sidx = (_CT * idx32[:, None, :]
          + jnp.arange(_CT, dtype=jnp.int32)[None, :, None])
  s4 = sidx.reshape(B // _NB, _NB, _CT, T).transpose(0, 2, 1, 3)
  main = jnp.pad(s4[:, :_CT - 1], ((0, 0), (0, 0), (0, 0), (0, 24 - T)))
  tail = s4[:, _CT - 1]
  sidx = jnp.concatenate(
      [main.reshape(B // _NB, -1), tail.reshape(B // _NB, -1)], axis=1
  ).reshape(-1)
  return _make_gather(B, T, V, D)(t128, sidx)


# speedup vs baseline: 1.0047x; 1.0047x over previous
import functools

import jax
import jax.numpy as jnp
from jax import lax
from jax.experimental import pallas as pl
from jax.experimental.pallas import tpu as pltpu
from jax.experimental.pallas import tpu_sc as plsc

_INFO = plsc.get_sparse_core_info()
_NC, _NS = _INFO.num_cores, _INFO.num_subcores
_NW = _NC * _NS
_NB = 2
_CT = 8


def _make_gather(B, T, V, D):
  b_per_w = B // _NW
  n_chunks = b_per_w // _NB
  seg = 24
  per_chunk = (_CT - 1) * _NB * seg + _NB * T
  n_per_w = n_chunks * per_chunk
  mesh = plsc.VectorSubcoreMesh(core_axis_name="c", subcore_axis_name="s")

  @functools.partial(
      pl.kernel,
      mesh=mesh,
      out_type=jax.ShapeDtypeStruct((B, T, D), jnp.float32),
      scratch_types=[
          pltpu.VMEM((2 * per_chunk,), jnp.int32),
          pltpu.VMEM((2, _NB, T, D), jnp.float32),
          pltpu.VMEM((2, _NB * T, 128), jnp.float32),
          pltpu.SemaphoreType.DMA,
          pltpu.SemaphoreType.DMA,
          pltpu.SemaphoreType.DMA,
          pltpu.SemaphoreType.DMA,
      ],
  )
  def gather_kernel(t128_hbm, sidx_hbm, out_hbm, sidx_v, rows_v, tail_v,
                    gsem0, gsem1, ssem0, ssem1):
    wid = lax.axis_index("s") * _NC + lax.axis_index("c")
    gsems = (gsem0, gsem1)
    ssems = (ssem0, ssem1)
    def fire_gathers(c, x, local, stage):
      if stage:
        pltpu.sync_copy(
            sidx_hbm.at[pl.ds(wid * n_per_w + (c // 2) * (2 * per_chunk),
                              2 * per_chunk)],
            sidx_v)
      o = local * per_chunk
      for bb in range(_NB):
        for ct in range(_CT - 1):
          pltpu.async_copy(
              t128_hbm.at[sidx_v.at[pl.ds(o + (ct * _NB + bb) * seg, T)]],
              rows_v.at[x, bb, pl.ds(0, T), pl.ds(ct * 128, 128)],
              gsems[x],
          )
      pltpu.async_copy(
          t128_hbm.at[sidx_v.at[pl.ds(o + (_CT - 1) * _NB * seg, _NB * T)]],
          tail_v.at[x], gsems[x])

    def wait_gathers(x):
      for bb in range(_NB):
        for ct in range(_CT - 1):
          pltpu.make_async_copy(
              t128_hbm.at[sidx_v.at[pl.ds((ct * _NB + bb) * seg, T)]],
              rows_v.at[x, bb, pl.ds(0, T), pl.ds(ct * 128, 128)],
              gsems[x],
          ).wait()
      pltpu.make_async_copy(
          t128_hbm.at[sidx_v.at[pl.ds(0, _NB * T)]], tail_v.at[x],
          gsems[x]).wait()

    def merge(x):
      def merge_row(t, _):
        for bb in range(_NB):
          for j in range(6):
            rows_v[x, bb, t, pl.ds(896 + 16 * j, 16)] = (
                tail_v[x, bb * T + t, pl.ds(16 * j, 16)])
          over = pl.multiple_of(t * 0 + 992, 16)
          rows_v[x, bb, t, pl.ds(over, 16)] = (
              tail_v[x, bb * T + t, pl.ds(96, 16)])
        return 0

      lax.fori_loop(0, T, merge_row, 0)

    def fire_scatter(c, x):
      b0 = wid * b_per_w + c * _NB
      pltpu.async_copy(rows_v.at[x], out_hbm.at[pl.ds(b0, _NB)], ssems[x])

    def wait_scatter(c, x):
      b0 = wid * b_per_w + c * _NB
      pltpu.make_async_copy(rows_v.at[x], out_hbm.at[pl.ds(b0, _NB)],
                            ssems[x]).wait()

    fire_gathers(0, 0, 0, True)
    wait_gathers(0)
    merge(0)
    fire_gathers(1, 1, 1, False)
    fire_scatter(0, 0)

    def body(p, _):
      j = 2 * p + 1
      wait_gathers(1)
      merge(1)
      wait_scatter(j - 1, 0)
      fire_gathers(j + 1, 0, 0, True)
      fire_scatter(j, 1)

      wait_gathers(0)
      merge(0)
      wait_scatter(j, 1)
      fire_gathers(j + 2, 1, 1, False)
      fire_scatter(j + 1, 0)
      return 0

    lax.fori_loop(0, (n_chunks - 2) // 2, body, 0)

    last = n_chunks - 1
    wait_gathers(1)
    merge(1)
    wait_scatter(last - 1, 0)
    fire_scatter(last, 1)
    wait_scatter(last, 1)

  return gather_kernel


def kernel(idx, table):
  B, T = idx.shape
  V, D = table.shape
  idx32 = idx.astype(jnp.int32)
  t128 = jnp.pad(table, ((0, 0), (0, _CT * 128 - D))).reshape(V * _CT, 128)
  sidx = (_CT * idx32[:, None, :]
          + jnp.arange(_CT, dtype=jnp.int32)[None, :, None])
  s4 = sidx.reshape(B // _NB, _NB, _CT, T).transpose(0, 2, 1, 3)
  main = jnp.pad(s4[:, :_CT - 1], ((0, 0), (0, 0), (0, 0), (0, 24 - T)))
  tail = s4[:, _CT - 1]
  sidx = jnp.concatenate(
      [main.reshape(B // _NB, -1), tail.reshape(B // _NB, -1)], axis=1
  ).reshape(-1)
  return _make_gather(B, T, V, D)(t128, sidx)
